# Initial kernel scaffold; baseline (speedup 1.0000x reference)
import jax
import jax.numpy as jnp
from jax.experimental import pallas as pl


def kernel(feat, edge_index, edge_weight, W):
    raise NotImplementedError("write your pallas kernel here")


# speedup vs baseline: 3.6728x; 3.6728x over previous
import functools

import jax
import jax.numpy as jnp
from jax import lax
from jax.experimental import pallas as pl
from jax.experimental.pallas import tpu as pltpu
from jax.experimental.pallas import tpu_sc as plsc

NC = 2
NS = 16
LANES = 16
CHUNK = 128


def _sc_aggregate(feat, src, dst, w, chunks_per_worker):
    n, d = feat.shape
    vregs_per_row = d // LANES
    pieces = -(-n // CHUNK)
    n_pad = pieces * CHUNK
    zsteps = -(-pieces // NS)

    mesh = plsc.VectorSubcoreMesh(core_axis_name="c", subcore_axis_name="s")

    @functools.partial(
        pl.kernel,
        out_type=jax.ShapeDtypeStruct((NC, n_pad, d), jnp.float32),
        mesh=mesh,
        scratch_types=[
            pltpu.VMEM((CHUNK,), jnp.int32),
            pltpu.VMEM((CHUNK,), jnp.int32),
            pltpu.VMEM((CHUNK,), jnp.float32),
            pltpu.VMEM((CHUNK, d), jnp.float32),
            pltpu.VMEM_SHARED((n_pad, d), jnp.float32),
            pltpu.SemaphoreType.DMA,
        ],
    )
    def agg(feat_hbm, src_hbm, dst_hbm, w_hbm, part_hbm,
            sidx, didx, wv, rows, acc, sem):
        cid = lax.axis_index("c")
        sid = lax.axis_index("s")
        wid = sid * NC + cid

        def zrow(i, _):
            e = i // vregs_per_row
            j = i % vregs_per_row
            rows[e, pl.ds(j * LANES, LANES)] = jnp.zeros((LANES,), jnp.float32)
            return 0
        lax.fori_loop(0, CHUNK * vregs_per_row, zrow, 0)
        for z in range(zsteps):
            p = sid + z * NS

            @pl.when(p < pieces)
            def _():
                pltpu.sync_copy(rows, acc.at[pl.ds(p * CHUNK, CHUNK)])
        plsc.subcore_barrier()

        def chunk_body(g, _):
            base = (wid * chunks_per_worker + g) * CHUNK
            pltpu.sync_copy(src_hbm.at[pl.ds(base, CHUNK)], sidx)
            pltpu.sync_copy(dst_hbm.at[pl.ds(base, CHUNK)], didx)
            pltpu.sync_copy(w_hbm.at[pl.ds(base, CHUNK)], wv)
            pltpu.async_copy(feat_hbm.at[sidx], rows, sem).wait()

            def scale(g, _):
                wvec = wv[pl.ds(g * LANES, LANES)]
                for l in range(LANES):
                    ws = jnp.full((LANES,), wvec[l], jnp.float32)
                    e = g * LANES + l
                    for j in range(vregs_per_row):
                        sl = pl.ds(j * LANES, LANES)
                        rows[e, sl] = rows[e, sl] * ws
                return 0
            lax.fori_loop(0, CHUNK // LANES, scale, 0)

            pltpu.sync_copy(rows, acc.at[didx], add=True)
            return 0

        lax.fori_loop(0, chunks_per_worker, chunk_body, 0)
        plsc.subcore_barrier()

        for z in range(zsteps):
            p = sid + z * NS

            @pl.when(p < pieces)
            def _():
                r0 = p * CHUNK
                pltpu.sync_copy(acc.at[pl.ds(r0, CHUNK)],
                                part_hbm.at[cid, pl.ds(r0, CHUNK)])

    return agg(feat, src, dst, w)


def _tc_finish(partial, W, n):
    nc, _, d = partial.shape
    d_out = W.shape[1]
    bn = 1000
    assert n % bn == 0

    def body(p_ref, w_ref, o_ref):
        s = p_ref[0] + p_ref[1]
        o_ref[...] = jnp.maximum(
            jnp.dot(s, w_ref[...], preferred_element_type=jnp.float32), 0.0)

    return pl.pallas_call(
        body,
        grid=(n // bn,),
        in_specs=[
            pl.BlockSpec((nc, bn, d), lambda i: (0, i, 0)),
            pl.BlockSpec((d, d_out), lambda i: (0, 0)),
        ],
        out_specs=pl.BlockSpec((bn, d_out), lambda i: (i, 0)),
        out_shape=jax.ShapeDtypeStruct((n, d_out), jnp.float32),
    )(partial, W)


def kernel(feat, edge_index, edge_weight, W):
    e = edge_weight.shape[0]
    per_worker = NC * NS * CHUNK
    chunks_per_worker = -(-e // per_worker)
    e_pad = per_worker * chunks_per_worker
    src = edge_index[0]
    dst = edge_index[1]
    w = edge_weight
    if e_pad > e:
        pad = e_pad - e
        src = jnp.concatenate([src, jnp.zeros((pad,), src.dtype)])
        dst = jnp.concatenate([dst, jnp.zeros((pad,), dst.dtype)])
        w = jnp.concatenate([w, jnp.zeros((pad,), w.dtype)])
    partial = _sc_aggregate(feat, src, dst, w, chunks_per_worker)
    return _tc_finish(partial, W, feat.shape[0])
